# Initial kernel scaffold; baseline (speedup 1.0000x reference)
import numpy as np
import jax
import jax.numpy as jnp
from jax import lax
from jax.experimental import pallas as pl
from jax.experimental.pallas import tpu as pltpu

_THRESH = float(-np.log(0.7))
_N_MIN = 131072
_IGNORE = 255

_B, _C, _H, _W = 8, 19, 512, 512
_CHUNK = 4096
_NJ = (_H * _W) // _CHUNK
_NPIX = _B * _H * _W

_SEL_ROWS = 128
_SEL_COLS = 1024
_NBLK = _NPIX // (_SEL_ROWS * _SEL_COLS)


def _ce_body(logits_ref, labels_ref, out_ref):
    x = logits_ref[0]
    lab = labels_ref[0]
    m = jnp.max(x, axis=0, keepdims=True)
    e = jnp.exp(x - m)
    s = jnp.sum(e, axis=0, keepdims=True)
    lse = jnp.log(s) + m
    cls = lax.broadcasted_iota(jnp.int32, (_C, _CHUNK), 0)
    safe_lab = jnp.where(lab == _IGNORE, 0, lab)
    picked = jnp.sum(jnp.where(cls == safe_lab, x, 0.0), axis=0, keepdims=True)
    loss = jnp.where(lab == _IGNORE, 0.0, lse - picked)
    out_ref[0] = loss


def _sel_body(loss_ref, out_ref, hist_s, st_s, fs_s):
    p = pl.program_id(0)
    b = pl.program_id(1)

    @pl.when((p == 0) & (b == 0))
    def _init():
        for j in range(16):
            hist_s[j] = 0
        st_s[0] = 0
        st_s[1] = _N_MIN
        st_s[2] = 0
        st_s[3] = 0
        fs_s[0] = 0.0
        fs_s[1] = 0.0

    @pl.when((p >= 1) & (b == 0))
    def _select_digit():
        remaining = st_s[1]
        acc = jnp.int32(0)
        dig = jnp.int32(0)
        newrem = remaining
        found = acc > jnp.int32(0)
        for j in range(15, -1, -1):
            hj = hist_s[j]
            hit = jnp.logical_and(jnp.logical_not(found), acc + hj >= remaining)
            dig = jnp.where(hit, jnp.int32(j), dig)
            newrem = jnp.where(hit, remaining - acc, newrem)
            found = jnp.logical_or(found, hit)
            acc = acc + hj
        st_s[0] = jnp.bitwise_or(lax.shift_left(st_s[0], 4), dig)
        st_s[1] = newrem
        for j in range(16):
            hist_s[j] = 0

    @pl.when(p <= 7)
    def _radix_count():
        x = loss_ref[...]
        u = lax.bitcast_convert_type(x, jnp.int32)
        shift = (7 - p) * 4
        us = lax.shift_right_logical(u, shift)
        d = jnp.bitwise_and(us, 15)
        hi = lax.shift_right_logical(us, 4)
        in_set = hi == st_s[0]
        base = jnp.where(in_set, d, 16)
        for j in range(16):
            hist_s[j] = hist_s[j] + jnp.sum((base == j).astype(jnp.int32))

    @pl.when(p == 8)
    def _stats():
        x = loss_ref[...]
        u = lax.bitcast_convert_type(x, jnp.int32)
        gt_thr = x > _THRESH
        fs_s[0] = fs_s[0] + jnp.sum(jnp.where(gt_thr, x, 0.0))
        st_s[2] = st_s[2] + jnp.sum(gt_thr.astype(jnp.int32))
        gt_t = u > st_s[0]
        fs_s[1] = fs_s[1] + jnp.sum(jnp.where(gt_t, x, 0.0))
        st_s[3] = st_s[3] + jnp.sum(gt_t.astype(jnp.int32))

    @pl.when((p == 8) & (b == _NBLK - 1))
    def _finalize():
        t = jnp.max(lax.bitcast_convert_type(
            jnp.full((1, 128), st_s[0], jnp.int32), jnp.float32))
        cnt_a = st_s[2]
        mean_a = fs_s[0] / jnp.maximum(cnt_a, 1).astype(jnp.float32)
        cnt_gt = st_s[3].astype(jnp.float32)
        sum_b = fs_s[1] + (jnp.float32(_N_MIN) - cnt_gt) * t
        mean_b = sum_b / jnp.float32(_N_MIN)
        out_ref[0, 0] = jnp.where(cnt_a > _N_MIN, mean_a, mean_b)


def _compute_loss(logits, labels):
    lg = logits.reshape(_B, _C, _H * _W)
    lb = labels.reshape(_B * _NJ, 1, _CHUNK)
    return pl.pallas_call(
        _ce_body,
        grid=(_B, _NJ),
        in_specs=[
            pl.BlockSpec((1, _C, _CHUNK), lambda b, j: (b, 0, j)),
            pl.BlockSpec((1, 1, _CHUNK), lambda b, j: (b * _NJ + j, 0, 0)),
        ],
        out_specs=pl.BlockSpec((1, 1, _CHUNK), lambda b, j: (b * _NJ + j, 0, 0)),
        out_shape=jax.ShapeDtypeStruct((_B * _NJ, 1, _CHUNK), jnp.float32),
        compiler_params=pltpu.CompilerParams(
            dimension_semantics=("arbitrary", "arbitrary")),
    )(lg, lb)


def _select(loss2d):
    return pl.pallas_call(
        _sel_body,
        grid=(9, _NBLK),
        in_specs=[pl.BlockSpec((_SEL_ROWS, _SEL_COLS), lambda p, b: (b, 0))],
        out_specs=pl.BlockSpec((1, 1), lambda p, b: (0, 0)),
        out_shape=jax.ShapeDtypeStruct((1, 1), jnp.float32),
        scratch_shapes=[
            pltpu.SMEM((16,), jnp.int32),
            pltpu.SMEM((4,), jnp.int32),
            pltpu.SMEM((2,), jnp.float32),
        ],
        compiler_params=pltpu.CompilerParams(
            dimension_semantics=("arbitrary", "arbitrary")),
    )(loss2d)


def kernel(logits, labels):
    loss = _compute_loss(logits, labels)
    loss2d = loss.reshape(_NBLK * _SEL_ROWS, _SEL_COLS)
    out = _select(loss2d)
    return out[0, 0]


# speedup vs baseline: 4.4004x; 4.4004x over previous
import numpy as np
import jax
import jax.numpy as jnp
from jax import lax
from jax.experimental import pallas as pl
from jax.experimental.pallas import tpu as pltpu

_THRESH = float(-np.log(0.7))
_N_MIN = 131072
_IGNORE = 255

_B, _C, _H, _W = 8, 19, 512, 512
_CHUNK = 4096
_NJ = (_H * _W) // _CHUNK
_NPIX = _B * _H * _W

_SEL_ROWS = 128
_SEL_COLS = 1024
_NBLK = _NPIX // (_SEL_ROWS * _SEL_COLS)


def _ce_body(logits_ref, labels_ref, out_ref):
    x = logits_ref[0]
    lab = labels_ref[0]
    m = jnp.max(x, axis=0, keepdims=True)
    e = jnp.exp(x - m)
    s = jnp.sum(e, axis=0, keepdims=True)
    lse = jnp.log(s) + m
    cls = lax.broadcasted_iota(jnp.int32, (_C, _CHUNK), 0)
    safe_lab = jnp.where(lab == _IGNORE, 0, lab)
    picked = jnp.sum(jnp.where(cls == safe_lab, x, 0.0), axis=0, keepdims=True)
    loss = jnp.where(lab == _IGNORE, 0.0, lse - picked)
    out_ref[0] = loss


def _sel_body(loss_ref, out_ref, hist_s, st_s, fs_s):
    p = pl.program_id(0)
    b = pl.program_id(1)

    @pl.when((p == 0) & (b == 0))
    def _init():
        for j in range(16):
            hist_s[j] = 0
        st_s[0] = 0
        st_s[1] = _N_MIN
        st_s[2] = 0
        st_s[3] = 0
        fs_s[0] = 0.0
        fs_s[1] = 0.0

    @pl.when((p >= 1) & (b == 0))
    def _select_digit():
        remaining = st_s[1]
        acc = jnp.int32(0)
        dig = jnp.int32(0)
        newrem = remaining
        found = acc > jnp.int32(0)
        for j in range(15, -1, -1):
            hj = hist_s[j]
            hit = jnp.logical_and(jnp.logical_not(found), acc + hj >= remaining)
            dig = jnp.where(hit, jnp.int32(j), dig)
            newrem = jnp.where(hit, remaining - acc, newrem)
            found = jnp.logical_or(found, hit)
            acc = acc + hj
        st_s[0] = jnp.bitwise_or(lax.shift_left(st_s[0], 4), dig)
        st_s[1] = newrem
        for j in range(16):
            hist_s[j] = 0

    @pl.when(p <= 7)
    def _radix_count():
        x = loss_ref[...]
        u = lax.bitcast_convert_type(x, jnp.int32)
        shift = (7 - p) * 4
        us = lax.shift_right_logical(u, shift)
        d = jnp.bitwise_and(us, 15)
        hi = lax.shift_right_logical(us, 4)
        in_set = hi == st_s[0]
        base = jnp.where(in_set, d, 16)
        for j in range(16):
            hist_s[j] = hist_s[j] + jnp.sum((base == j).astype(jnp.int32))

    @pl.when(p == 8)
    def _stats():
        x = loss_ref[...]
        u = lax.bitcast_convert_type(x, jnp.int32)
        gt_thr = x > _THRESH
        fs_s[0] = fs_s[0] + jnp.sum(jnp.where(gt_thr, x, 0.0))
        st_s[2] = st_s[2] + jnp.sum(gt_thr.astype(jnp.int32))
        gt_t = u > st_s[0]
        fs_s[1] = fs_s[1] + jnp.sum(jnp.where(gt_t, x, 0.0))
        st_s[3] = st_s[3] + jnp.sum(gt_t.astype(jnp.int32))

    @pl.when((p == 8) & (b == _NBLK - 1))
    def _finalize():
        t = jnp.max(lax.bitcast_convert_type(
            jnp.full((1, 128), st_s[0], jnp.int32), jnp.float32))
        cnt_a = st_s[2]
        mean_a = fs_s[0] / jnp.maximum(cnt_a, 1).astype(jnp.float32)
        cnt_gt = st_s[3].astype(jnp.float32)
        sum_b = fs_s[1] + (jnp.float32(_N_MIN) - cnt_gt) * t
        mean_b = sum_b / jnp.float32(_N_MIN)
        out_ref[...] = jnp.full((1, 128), jnp.where(cnt_a > _N_MIN, mean_a, mean_b),
                                jnp.float32)


def _compute_loss(logits, labels):
    lg = logits.reshape(_B, _C, _H * _W)
    lb = labels.reshape(_B * _NJ, 1, _CHUNK)
    return pl.pallas_call(
        _ce_body,
        grid=(_B, _NJ),
        in_specs=[
            pl.BlockSpec((1, _C, _CHUNK), lambda b, j: (b, 0, j)),
            pl.BlockSpec((1, 1, _CHUNK), lambda b, j: (b * _NJ + j, 0, 0)),
        ],
        out_specs=pl.BlockSpec((1, 1, _CHUNK), lambda b, j: (b * _NJ + j, 0, 0)),
        out_shape=jax.ShapeDtypeStruct((_B * _NJ, 1, _CHUNK), jnp.float32),
        compiler_params=pltpu.CompilerParams(
            dimension_semantics=("arbitrary", "arbitrary")),
    )(lg, lb)


def _select(loss2d):
    return pl.pallas_call(
        _sel_body,
        grid=(9, _NBLK),
        in_specs=[pl.BlockSpec((_SEL_ROWS, _SEL_COLS), lambda p, b: (b, 0))],
        out_specs=pl.BlockSpec((1, 128), lambda p, b: (0, 0)),
        out_shape=jax.ShapeDtypeStruct((1, 128), jnp.float32),
        scratch_shapes=[
            pltpu.SMEM((16,), jnp.int32),
            pltpu.SMEM((4,), jnp.int32),
            pltpu.SMEM((2,), jnp.float32),
        ],
        compiler_params=pltpu.CompilerParams(
            dimension_semantics=("arbitrary", "arbitrary")),
    )(loss2d)


def kernel(logits, labels):
    loss = _compute_loss(logits, labels)
    loss2d = loss.reshape(_NBLK * _SEL_ROWS, _SEL_COLS)
    out = _select(loss2d)
    return out[0, 0]


# speedup vs baseline: 6.4801x; 1.4726x over previous
import numpy as np
import jax
import jax.numpy as jnp
from jax import lax
from jax.experimental import pallas as pl
from jax.experimental.pallas import tpu as pltpu

_THRESH = float(-np.log(0.7))
_N_MIN = 131072
_IGNORE = 255

_B, _C, _H, _W = 8, 19, 512, 512
_CROWS = 8
_NJ = (_H * _W) // (_CROWS * 1024)
_NPIX = _B * _H * _W

_SEL_ROWS = 128
_SEL_COLS = 1024
_NBLK = _NPIX // (_SEL_ROWS * _SEL_COLS)


def _ce_body(logits_ref, labels_ref, out_ref):
    x = logits_ref[0]
    lab = labels_ref[0]
    m = jnp.max(x, axis=0)
    e = jnp.exp(x - m[None])
    s = jnp.sum(e, axis=0)
    lse = jnp.log(s) + m
    cls = lax.broadcasted_iota(jnp.int32, (_C, _CROWS, 1024), 0)
    safe_lab = jnp.where(lab == _IGNORE, 0, lab)
    picked = jnp.sum(jnp.where(cls == safe_lab[None], x, 0.0), axis=0)
    loss = jnp.where(lab == _IGNORE, 0.0, lse - picked)
    out_ref[...] = loss


def _sel_body(loss_ref, out_ref, hist_s, st_s, fs_s):
    p = pl.program_id(0)
    b = pl.program_id(1)

    @pl.when((p == 0) & (b == 0))
    def _init():
        for j in range(16):
            hist_s[j] = 0
        st_s[0] = 0
        st_s[1] = _N_MIN
        st_s[2] = 0
        st_s[3] = 0
        fs_s[0] = 0.0
        fs_s[1] = 0.0

    @pl.when((p >= 1) & (b == 0))
    def _select_digit():
        remaining = st_s[1]
        acc = jnp.int32(0)
        dig = jnp.int32(0)
        newrem = remaining
        found = acc > jnp.int32(0)
        for j in range(15, -1, -1):
            hj = hist_s[j]
            hit = jnp.logical_and(jnp.logical_not(found), acc + hj >= remaining)
            dig = jnp.where(hit, jnp.int32(j), dig)
            newrem = jnp.where(hit, remaining - acc, newrem)
            found = jnp.logical_or(found, hit)
            acc = acc + hj
        st_s[0] = jnp.bitwise_or(lax.shift_left(st_s[0], 4), dig)
        st_s[1] = newrem
        for j in range(16):
            hist_s[j] = 0

    @pl.when(p <= 7)
    def _radix_count():
        x = loss_ref[...]
        u = lax.bitcast_convert_type(x, jnp.int32)
        shift = (7 - p) * 4
        us = lax.shift_right_logical(u, shift)
        d = jnp.bitwise_and(us, 15)
        hi = lax.shift_right_logical(us, 4)
        in_set = hi == st_s[0]
        base = jnp.where(in_set, d, 16)
        for j in range(16):
            hist_s[j] = hist_s[j] + jnp.sum((base == j).astype(jnp.int32))

    @pl.when(p == 8)
    def _stats():
        x = loss_ref[...]
        u = lax.bitcast_convert_type(x, jnp.int32)
        gt_thr = x > _THRESH
        fs_s[0] = fs_s[0] + jnp.sum(jnp.where(gt_thr, x, 0.0))
        st_s[2] = st_s[2] + jnp.sum(gt_thr.astype(jnp.int32))
        gt_t = u > st_s[0]
        fs_s[1] = fs_s[1] + jnp.sum(jnp.where(gt_t, x, 0.0))
        st_s[3] = st_s[3] + jnp.sum(gt_t.astype(jnp.int32))

    @pl.when((p == 8) & (b == _NBLK - 1))
    def _finalize():
        t = jnp.max(lax.bitcast_convert_type(
            jnp.full((1, 128), st_s[0], jnp.int32), jnp.float32))
        cnt_a = st_s[2]
        mean_a = fs_s[0] / jnp.maximum(cnt_a, 1).astype(jnp.float32)
        cnt_gt = st_s[3].astype(jnp.float32)
        sum_b = fs_s[1] + (jnp.float32(_N_MIN) - cnt_gt) * t
        mean_b = sum_b / jnp.float32(_N_MIN)
        out_ref[...] = jnp.full((1, 128), jnp.where(cnt_a > _N_MIN, mean_a, mean_b),
                                jnp.float32)


def _compute_loss(logits, labels):
    lg = logits.reshape(_B, _C, _NJ * _CROWS, 1024)
    lb = labels.reshape(_B, _NJ * _CROWS, 1024)
    return pl.pallas_call(
        _ce_body,
        grid=(_B, _NJ),
        in_specs=[
            pl.BlockSpec((1, _C, _CROWS, 1024), lambda b, j: (b, 0, j, 0)),
            pl.BlockSpec((1, _CROWS, 1024), lambda b, j: (b, j, 0)),
        ],
        out_specs=pl.BlockSpec((_CROWS, 1024), lambda b, j: (b * _NJ + j, 0)),
        out_shape=jax.ShapeDtypeStruct((_B * _NJ * _CROWS, 1024), jnp.float32),
        compiler_params=pltpu.CompilerParams(
            dimension_semantics=("arbitrary", "arbitrary")),
    )(lg, lb)


def _select(loss2d):
    return pl.pallas_call(
        _sel_body,
        grid=(9, _NBLK),
        in_specs=[pl.BlockSpec((_SEL_ROWS, _SEL_COLS), lambda p, b: (b, 0))],
        out_specs=pl.BlockSpec((1, 128), lambda p, b: (0, 0)),
        out_shape=jax.ShapeDtypeStruct((1, 128), jnp.float32),
        scratch_shapes=[
            pltpu.SMEM((16,), jnp.int32),
            pltpu.SMEM((4,), jnp.int32),
            pltpu.SMEM((2,), jnp.float32),
        ],
        compiler_params=pltpu.CompilerParams(
            dimension_semantics=("arbitrary", "arbitrary")),
    )(loss2d)


def kernel(logits, labels):
    loss = _compute_loss(logits, labels)
    out = _select(loss)
    return out[0, 0]
